# baseline (speedup 1.0000x reference)
import functools
import math

import jax
import jax.numpy as jnp
from jax import lax
from jax.experimental import pallas as pl
from jax.experimental.pallas import tpu as pltpu
from jax.experimental.pallas import tpu_sc as plsc

_D = 64
_SCALE = math.sqrt(_D)
_NC, _NS = 2, 16
_NW = _NC * _NS
_CH = 512
_IDXW = 128
_CHK = _CH // _IDXW


def _emb_body(n_chunks, x_hbm, lut_hbm, out_hbm,
              idx_a, idx_b, rows_a, rows_b, sem_a, sem_b):
  wid = lax.axis_index("s") * _NC + lax.axis_index("c")
  rows_per_w = n_chunks * _CH
  idx_rows_per_ch = _CHK
  base_idx_row = wid * n_chunks * idx_rows_per_ch
  base_out = wid * rows_per_w

  idx_bufs = (idx_a, idx_b)
  rows_bufs = (rows_a, rows_b)
  sems = (sem_a, sem_b)

  def stage_and_fire(g, b):
    pltpu.sync_copy(x_hbm.at[pl.ds(base_idx_row + g * idx_rows_per_ch,
                                   idx_rows_per_ch)], idx_bufs[b])
    for k in range(_CHK):
      pltpu.async_copy(lut_hbm.at[idx_bufs[b].at[k]],
                       rows_bufs[b].at[pl.ds(k * _IDXW, _IDXW)], sems[b])

  def drain(b):
    for k in range(_CHK):
      pltpu.make_async_copy(lut_hbm.at[pl.ds(0, _IDXW)],
                            rows_bufs[b].at[pl.ds(k * _IDXW, _IDXW)],
                            sems[b]).wait()

  def scale(b):
    buf = rows_bufs[b]

    @pl.loop(0, _CH, step=4)
    def _(r):
      for rr in range(4):
        for j in range(_D // 16):
          sl = (r + rr, pl.ds(j * 16, 16))
          buf[sl] = buf[sl] * _SCALE

  stage_and_fire(0, 0)

  @pl.loop(0, n_chunks, step=2)
  def _(gbase):
    for b in range(2):
      g = gbase + b

      @pl.when(g + 1 < n_chunks)
      def _():
        stage_and_fire(g + 1, 1 - b)

      drain(b)
      scale(b)
      pltpu.sync_copy(rows_bufs[b],
                      out_hbm.at[pl.ds(base_out + g * _CH, _CH)])


@jax.jit
def kernel(x, lut):
  n_total = x.shape[0] * x.shape[1]
  assert n_total % (_NW * _CH) == 0
  n_chunks = n_total // (_NW * _CH)
  x2d = x.reshape(n_total // _IDXW, _IDXW).astype(jnp.int32)

  mesh = plsc.VectorSubcoreMesh(core_axis_name="c", subcore_axis_name="s",
                                num_cores=_NC, num_subcores=_NS)
  out = pl.kernel(
      functools.partial(_emb_body, n_chunks),
      out_type=jax.ShapeDtypeStruct((n_total, _D), jnp.float32),
      mesh=mesh,
      compiler_params=pltpu.CompilerParams(use_tc_tiling_on_sc=False),
      scratch_types=[
          pltpu.VMEM((_CHK, _IDXW), jnp.int32),
          pltpu.VMEM((_CHK, _IDXW), jnp.int32),
          pltpu.VMEM((_CH, _D), jnp.float32),
          pltpu.VMEM((_CH, _D), jnp.float32),
          pltpu.SemaphoreType.DMA,
          pltpu.SemaphoreType.DMA,
      ],
  )(x2d, lut)
  return out.reshape(x.shape[0], x.shape[1], _D)
